# Initial kernel scaffold; baseline (speedup 1.0000x reference)
import jax
import jax.numpy as jnp
from jax.experimental import pallas as pl


def kernel(x_user, x_item, edge_index_u2i, edge_index_i2u, W1_l_ui, b1_ui, W1_r_ui, W1_l_iu, b1_iu, W1_r_iu, W2_l_ui, b2_ui, W2_r_ui, W2_l_iu, b2_iu, W2_r_iu):
    raise NotImplementedError("write your pallas kernel here")


# speedup vs baseline: 2.6793x; 2.6793x over previous
import jax
import jax.numpy as jnp
from jax import lax
from jax.experimental import pallas as pl
from jax.experimental.pallas import tpu as pltpu
from jax.experimental.pallas import tpu_sc as plsc

N = 10000
F = 128
N_PAD = 10112
NS = 16
NC = 2
CHUNK = 128
CHUNKS_PER_TILE = 80
E_PAD = NS * CHUNKS_PER_TILE * CHUNK
ROWS_PER_TILE = N_PAD // NS
BM = 400


def _mm_body(x_ref, w_ref, tab_ref, r_ref):
    z = jnp.dot(x_ref[...], w_ref[...], preferred_element_type=jnp.float32)
    tab_ref[...] = z[:, :F]
    r_ref[...] = z[:, F:]


def _mm(x, w_aug):
    k = x.shape[1]
    return pl.pallas_call(
        _mm_body,
        grid=(N // BM,),
        in_specs=[
            pl.BlockSpec((BM, k), lambda i: (i, 0)),
            pl.BlockSpec((k, 2 * F), lambda i: (0, 0)),
        ],
        out_specs=[
            pl.BlockSpec((BM, F), lambda i: (i, 0)),
            pl.BlockSpec((BM, F), lambda i: (i, 0)),
        ],
        out_shape=[
            jax.ShapeDtypeStruct((N, F), jnp.float32),
            jax.ShapeDtypeStruct((N, F), jnp.float32),
        ],
    )(x, w_aug)


def _epi_body_relu(acc_ref, cnt_ref, r_ref, b_ref, o_ref):
    _epi_common(acc_ref, cnt_ref, r_ref, b_ref, o_ref, True)


def _epi_body_lin(acc_ref, cnt_ref, r_ref, b_ref, o_ref):
    _epi_common(acc_ref, cnt_ref, r_ref, b_ref, o_ref, False)


def _epi_common(acc_ref, cnt_ref, r_ref, b_ref, o_ref, relu):
    cnt = jnp.maximum(cnt_ref[...], 1.0)
    out = acc_ref[...] / cnt + b_ref[...] + r_ref[...]
    if relu:
        out = jnp.maximum(out, 0.0)
    o_ref[...] = out


def _epi(acc, cnt, r, b, relu):
    body = _epi_body_relu if relu else _epi_body_lin
    return pl.pallas_call(
        body,
        grid=(N // BM,),
        in_specs=[
            pl.BlockSpec((BM, F), lambda i: (i, 0)),
            pl.BlockSpec((BM, F), lambda i: (i, 0)),
            pl.BlockSpec((BM, F), lambda i: (i, 0)),
            pl.BlockSpec((1, F), lambda i: (0, 0)),
        ],
        out_specs=pl.BlockSpec((BM, F), lambda i: (i, 0)),
        out_shape=jax.ShapeDtypeStruct((N, F), jnp.float32),
    )(acc, cnt, r, b.reshape(1, F))


def _sc_scatter_body(tab_u, tab_i, s_ui, d_ui, s_iu, d_iu, zinit,
                     out_i, out_u, src_v, dst_v, rows_v, acc_sh, sem):
    cid = lax.axis_index("c")
    sid = lax.axis_index("s")
    row0 = sid * ROWS_PER_TILE
    pltpu.sync_copy(zinit, acc_sh.at[pl.ds(row0, ROWS_PER_TILE)])
    plsc.subcore_barrier()

    @pl.when(cid == 0)
    def _():
        def body(g, carry):
            base = sid * (CHUNKS_PER_TILE * CHUNK) + g * CHUNK
            pltpu.sync_copy(s_ui.at[pl.ds(base, CHUNK)], src_v)
            pltpu.sync_copy(d_ui.at[pl.ds(base, CHUNK)], dst_v)
            pltpu.async_copy(tab_u.at[src_v], rows_v, sem).wait()
            pltpu.sync_copy(rows_v, acc_sh.at[dst_v], add=True)
            return carry
        lax.fori_loop(0, CHUNKS_PER_TILE, body, 0)

    @pl.when(cid == 1)
    def _():
        def body(g, carry):
            base = sid * (CHUNKS_PER_TILE * CHUNK) + g * CHUNK
            pltpu.sync_copy(s_iu.at[pl.ds(base, CHUNK)], src_v)
            pltpu.sync_copy(d_iu.at[pl.ds(base, CHUNK)], dst_v)
            pltpu.async_copy(tab_i.at[src_v], rows_v, sem).wait()
            pltpu.sync_copy(rows_v, acc_sh.at[dst_v], add=True)
            return carry
        lax.fori_loop(0, CHUNKS_PER_TILE, body, 0)

    plsc.subcore_barrier()

    @pl.when(cid == 0)
    def _():
        pltpu.sync_copy(acc_sh.at[pl.ds(row0, ROWS_PER_TILE)],
                        out_i.at[pl.ds(row0, ROWS_PER_TILE)])

    @pl.when(cid == 1)
    def _():
        pltpu.sync_copy(acc_sh.at[pl.ds(row0, ROWS_PER_TILE)],
                        out_u.at[pl.ds(row0, ROWS_PER_TILE)])


def _sc_scatter(tab_u, tab_i, s_ui, d_ui, s_iu, d_iu, zinit):
    mesh = plsc.VectorSubcoreMesh(core_axis_name="c", subcore_axis_name="s",
                                  num_cores=NC, num_subcores=NS)
    f = pl.kernel(
        _sc_scatter_body,
        out_type=(jax.ShapeDtypeStruct((N_PAD, F), jnp.float32),
                  jax.ShapeDtypeStruct((N_PAD, F), jnp.float32)),
        mesh=mesh,
        scratch_types=[
            pltpu.VMEM((CHUNK,), jnp.int32),
            pltpu.VMEM((CHUNK,), jnp.int32),
            pltpu.VMEM((CHUNK, F), jnp.float32),
            pltpu.VMEM_SHARED((N_PAD, F), jnp.float32),
            pltpu.SemaphoreType.DMA,
        ],
    )
    return f(tab_u, tab_i, s_ui, d_ui, s_iu, d_iu, zinit)


def _sc_count_body(d_ui, d_iu, ones_rows, zinit, out_i, out_u,
                   dst_v, rows_v, acc_sh):
    cid = lax.axis_index("c")
    sid = lax.axis_index("s")
    row0 = sid * ROWS_PER_TILE
    pltpu.sync_copy(zinit, acc_sh.at[pl.ds(row0, ROWS_PER_TILE)])
    pltpu.sync_copy(ones_rows, rows_v)
    plsc.subcore_barrier()

    @pl.when(cid == 0)
    def _():
        def body(g, carry):
            base = sid * (CHUNKS_PER_TILE * CHUNK) + g * CHUNK
            pltpu.sync_copy(d_ui.at[pl.ds(base, CHUNK)], dst_v)
            pltpu.sync_copy(rows_v, acc_sh.at[dst_v], add=True)
            return carry
        lax.fori_loop(0, CHUNKS_PER_TILE, body, 0)

    @pl.when(cid == 1)
    def _():
        def body(g, carry):
            base = sid * (CHUNKS_PER_TILE * CHUNK) + g * CHUNK
            pltpu.sync_copy(d_iu.at[pl.ds(base, CHUNK)], dst_v)
            pltpu.sync_copy(rows_v, acc_sh.at[dst_v], add=True)
            return carry
        lax.fori_loop(0, CHUNKS_PER_TILE, body, 0)

    plsc.subcore_barrier()

    @pl.when(cid == 0)
    def _():
        pltpu.sync_copy(acc_sh.at[pl.ds(row0, ROWS_PER_TILE)],
                        out_i.at[pl.ds(row0, ROWS_PER_TILE)])

    @pl.when(cid == 1)
    def _():
        pltpu.sync_copy(acc_sh.at[pl.ds(row0, ROWS_PER_TILE)],
                        out_u.at[pl.ds(row0, ROWS_PER_TILE)])


def _sc_count(d_ui, d_iu, ones_rows, zinit):
    mesh = plsc.VectorSubcoreMesh(core_axis_name="c", subcore_axis_name="s",
                                  num_cores=NC, num_subcores=NS)
    f = pl.kernel(
        _sc_count_body,
        out_type=(jax.ShapeDtypeStruct((N_PAD, F), jnp.float32),
                  jax.ShapeDtypeStruct((N_PAD, F), jnp.float32)),
        mesh=mesh,
        scratch_types=[
            pltpu.VMEM((CHUNK,), jnp.int32),
            pltpu.VMEM((CHUNK, F), jnp.float32),
            pltpu.VMEM_SHARED((N_PAD, F), jnp.float32),
        ],
    )
    return f(d_ui, d_iu, ones_rows, zinit)


def _pad_edges(ei):
    src = ei[0].astype(jnp.int32)
    dst = ei[1].astype(jnp.int32)
    pad = E_PAD - src.shape[0]
    src = jnp.concatenate([src, jnp.zeros((pad,), jnp.int32)])
    dst = jnp.concatenate([dst, jnp.full((pad,), N, jnp.int32)])
    return src, dst


def kernel(x_user, x_item, edge_index_u2i, edge_index_i2u,
           W1_l_ui, b1_ui, W1_r_ui, W1_l_iu, b1_iu, W1_r_iu,
           W2_l_ui, b2_ui, W2_r_ui, W2_l_iu, b2_iu, W2_r_iu):
    s_ui, d_ui = _pad_edges(edge_index_u2i)
    s_iu, d_iu = _pad_edges(edge_index_i2u)
    zinit = jnp.zeros((ROWS_PER_TILE, F), jnp.float32)
    ones_rows = jnp.ones((CHUNK, F), jnp.float32)

    cnt_i, cnt_u = _sc_count(d_ui, d_iu, ones_rows, zinit)

    tab_u1, r_user1 = _mm(x_user, jnp.concatenate([W1_l_ui, W1_r_iu], axis=1))
    tab_i1, r_item1 = _mm(x_item, jnp.concatenate([W1_l_iu, W1_r_ui], axis=1))
    acc_i1, acc_u1 = _sc_scatter(tab_u1, tab_i1, s_ui, d_ui, s_iu, d_iu, zinit)
    x_item1 = _epi(acc_i1[:N], cnt_i[:N], r_item1, b1_ui, relu=True)
    x_user1 = _epi(acc_u1[:N], cnt_u[:N], r_user1, b1_iu, relu=True)

    tab_u2, r_user2 = _mm(x_user1, jnp.concatenate([W2_l_ui, W2_r_iu], axis=1))
    tab_i2, r_item2 = _mm(x_item1, jnp.concatenate([W2_l_iu, W2_r_ui], axis=1))
    acc_i2, acc_u2 = _sc_scatter(tab_u2, tab_i2, s_ui, d_ui, s_iu, d_iu, zinit)
    x_item2 = _epi(acc_i2[:N], cnt_i[:N], r_item2, b2_ui, relu=False)
    x_user2 = _epi(acc_u2[:N], cnt_u[:N], r_user2, b2_iu, relu=False)
    return (x_user2, x_item2)


# speedup vs baseline: 3.0684x; 1.1452x over previous
import jax
import jax.numpy as jnp
from jax import lax
from jax.experimental import pallas as pl
from jax.experimental.pallas import tpu as pltpu
from jax.experimental.pallas import tpu_sc as plsc

N = 10000
F = 128
N_PAD = 10112
NS = 16
NC = 2
CHUNK = 128
CHUNKS_PER_TILE = 80
E_PAD = NS * CHUNKS_PER_TILE * CHUNK
ROWS_PER_TILE = N_PAD // NS
BM = 400


def _mm_body(x_ref, w_ref, tab_ref, r_ref):
    z = jnp.dot(x_ref[...], w_ref[...], preferred_element_type=jnp.float32)
    tab_ref[...] = z[:, :F]
    r_ref[...] = z[:, F:]


def _mm(x, w_aug):
    k = x.shape[1]
    return pl.pallas_call(
        _mm_body,
        grid=(N // BM,),
        in_specs=[
            pl.BlockSpec((BM, k), lambda i: (i, 0)),
            pl.BlockSpec((k, 2 * F), lambda i: (0, 0)),
        ],
        out_specs=[
            pl.BlockSpec((BM, F), lambda i: (i, 0)),
            pl.BlockSpec((BM, F), lambda i: (i, 0)),
        ],
        out_shape=[
            jax.ShapeDtypeStruct((N, F), jnp.float32),
            jax.ShapeDtypeStruct((N, F), jnp.float32),
        ],
    )(x, w_aug)


def _epi_body_relu(acc_ref, cnt_ref, r_ref, b_ref, o_ref):
    _epi_common(acc_ref, cnt_ref, r_ref, b_ref, o_ref, True)


def _epi_body_lin(acc_ref, cnt_ref, r_ref, b_ref, o_ref):
    _epi_common(acc_ref, cnt_ref, r_ref, b_ref, o_ref, False)


def _epi_common(acc_ref, cnt_ref, r_ref, b_ref, o_ref, relu):
    cnt = jnp.maximum(cnt_ref[...], 1.0)
    out = acc_ref[...] / cnt + b_ref[...] + r_ref[...]
    if relu:
        out = jnp.maximum(out, 0.0)
    o_ref[...] = out


def _epi(acc, cnt, r, b, relu):
    body = _epi_body_relu if relu else _epi_body_lin
    return pl.pallas_call(
        body,
        grid=(N // BM,),
        in_specs=[
            pl.BlockSpec((BM, F), lambda i: (i, 0)),
            pl.BlockSpec((BM, F), lambda i: (i, 0)),
            pl.BlockSpec((BM, F), lambda i: (i, 0)),
            pl.BlockSpec((1, F), lambda i: (0, 0)),
        ],
        out_specs=pl.BlockSpec((BM, F), lambda i: (i, 0)),
        out_shape=jax.ShapeDtypeStruct((N, F), jnp.float32),
    )(acc, cnt, r, b.reshape(1, F))


NBUF = 2
OUTER = CHUNKS_PER_TILE // NBUF
NBUF_C = 5
OUTER_C = CHUNKS_PER_TILE // NBUF_C


def _sc_scatter_body(tab_u, tab_i, s_ui, d_ui, s_iu, d_iu, zinit,
                     out_i, out_u, src_b, dst_b, rows_b, acc_sh,
                     sem_i, sem_g):
    cid = lax.axis_index("c")
    sid = lax.axis_index("s")
    row0 = sid * ROWS_PER_TILE
    pltpu.sync_copy(zinit, acc_sh.at[pl.ds(row0, ROWS_PER_TILE)])
    plsc.subcore_barrier()

    def run(tab, s_hbm, d_hbm):
        def outer(g0, carry):
            hs = []
            for b in range(NBUF):
                base = sid * (CHUNKS_PER_TILE * CHUNK) \
                    + (g0 * NBUF + b) * CHUNK
                h1 = pltpu.async_copy(s_hbm.at[pl.ds(base, CHUNK)],
                                      src_b[b], sem_i[b])
                h2 = pltpu.async_copy(d_hbm.at[pl.ds(base, CHUNK)],
                                      dst_b[b], sem_i[b])
                hs.append((h1, h2))
            gs = []
            for b in range(NBUF):
                hs[b][0].wait()
                hs[b][1].wait()
                gs.append(pltpu.async_copy(tab.at[src_b[b]],
                                           rows_b[b], sem_g[b]))
            for b in range(NBUF):
                gs[b].wait()
                pltpu.sync_copy(rows_b[b], acc_sh.at[dst_b[b]], add=True)
            return carry
        lax.fori_loop(0, OUTER, outer, 0)

    @pl.when(cid == 0)
    def _():
        run(tab_u, s_ui, d_ui)

    @pl.when(cid == 1)
    def _():
        run(tab_i, s_iu, d_iu)

    plsc.subcore_barrier()

    @pl.when(cid == 0)
    def _():
        pltpu.sync_copy(acc_sh.at[pl.ds(row0, ROWS_PER_TILE)],
                        out_i.at[pl.ds(row0, ROWS_PER_TILE)])

    @pl.when(cid == 1)
    def _():
        pltpu.sync_copy(acc_sh.at[pl.ds(row0, ROWS_PER_TILE)],
                        out_u.at[pl.ds(row0, ROWS_PER_TILE)])


def _sc_scatter(tab_u, tab_i, s_ui, d_ui, s_iu, d_iu, zinit):
    mesh = plsc.VectorSubcoreMesh(core_axis_name="c", subcore_axis_name="s",
                                  num_cores=NC, num_subcores=NS)
    f = pl.kernel(
        _sc_scatter_body,
        out_type=(jax.ShapeDtypeStruct((N_PAD, F), jnp.float32),
                  jax.ShapeDtypeStruct((N_PAD, F), jnp.float32)),
        mesh=mesh,
        scratch_types=[
            [pltpu.VMEM((CHUNK,), jnp.int32) for _ in range(NBUF)],
            [pltpu.VMEM((CHUNK,), jnp.int32) for _ in range(NBUF)],
            [pltpu.VMEM((CHUNK, F), jnp.float32) for _ in range(NBUF)],
            pltpu.VMEM_SHARED((N_PAD, F), jnp.float32),
            [pltpu.SemaphoreType.DMA for _ in range(NBUF)],
            [pltpu.SemaphoreType.DMA for _ in range(NBUF)],
        ],
    )
    return f(tab_u, tab_i, s_ui, d_ui, s_iu, d_iu, zinit)


def _sc_count_body(d_ui, d_iu, ones_rows, zinit, out_i, out_u,
                   dst_b, rows_v, acc_sh, sem_i):
    cid = lax.axis_index("c")
    sid = lax.axis_index("s")
    row0 = sid * ROWS_PER_TILE
    pltpu.sync_copy(zinit, acc_sh.at[pl.ds(row0, ROWS_PER_TILE)])
    pltpu.sync_copy(ones_rows, rows_v)
    plsc.subcore_barrier()

    def run(d_hbm):
        def outer(g0, carry):
            hs = []
            for b in range(NBUF_C):
                base = sid * (CHUNKS_PER_TILE * CHUNK) \
                    + (g0 * NBUF_C + b) * CHUNK
                hs.append(pltpu.async_copy(d_hbm.at[pl.ds(base, CHUNK)],
                                           dst_b[b], sem_i[b]))
            for b in range(NBUF_C):
                hs[b].wait()
                pltpu.sync_copy(rows_v, acc_sh.at[dst_b[b]], add=True)
            return carry
        lax.fori_loop(0, OUTER_C, outer, 0)

    @pl.when(cid == 0)
    def _():
        run(d_ui)

    @pl.when(cid == 1)
    def _():
        run(d_iu)

    plsc.subcore_barrier()

    @pl.when(cid == 0)
    def _():
        pltpu.sync_copy(acc_sh.at[pl.ds(row0, ROWS_PER_TILE)],
                        out_i.at[pl.ds(row0, ROWS_PER_TILE)])

    @pl.when(cid == 1)
    def _():
        pltpu.sync_copy(acc_sh.at[pl.ds(row0, ROWS_PER_TILE)],
                        out_u.at[pl.ds(row0, ROWS_PER_TILE)])


def _sc_count(d_ui, d_iu, ones_rows, zinit):
    mesh = plsc.VectorSubcoreMesh(core_axis_name="c", subcore_axis_name="s",
                                  num_cores=NC, num_subcores=NS)
    f = pl.kernel(
        _sc_count_body,
        out_type=(jax.ShapeDtypeStruct((N_PAD, F), jnp.float32),
                  jax.ShapeDtypeStruct((N_PAD, F), jnp.float32)),
        mesh=mesh,
        scratch_types=[
            [pltpu.VMEM((CHUNK,), jnp.int32) for _ in range(NBUF_C)],
            pltpu.VMEM((CHUNK, F), jnp.float32),
            pltpu.VMEM_SHARED((N_PAD, F), jnp.float32),
            [pltpu.SemaphoreType.DMA for _ in range(NBUF_C)],
        ],
    )
    return f(d_ui, d_iu, ones_rows, zinit)


def _pad_edges(ei):
    src = ei[0].astype(jnp.int32)
    dst = ei[1].astype(jnp.int32)
    pad = E_PAD - src.shape[0]
    src = jnp.concatenate([src, jnp.zeros((pad,), jnp.int32)])
    dst = jnp.concatenate([dst, jnp.full((pad,), N, jnp.int32)])
    return src, dst


def kernel(x_user, x_item, edge_index_u2i, edge_index_i2u,
           W1_l_ui, b1_ui, W1_r_ui, W1_l_iu, b1_iu, W1_r_iu,
           W2_l_ui, b2_ui, W2_r_ui, W2_l_iu, b2_iu, W2_r_iu):
    s_ui, d_ui = _pad_edges(edge_index_u2i)
    s_iu, d_iu = _pad_edges(edge_index_i2u)
    zinit = jnp.zeros((ROWS_PER_TILE, F), jnp.float32)
    ones_rows = jnp.ones((CHUNK, F), jnp.float32)

    cnt_i, cnt_u = _sc_count(d_ui, d_iu, ones_rows, zinit)

    tab_u1, r_user1 = _mm(x_user, jnp.concatenate([W1_l_ui, W1_r_iu], axis=1))
    tab_i1, r_item1 = _mm(x_item, jnp.concatenate([W1_l_iu, W1_r_ui], axis=1))
    acc_i1, acc_u1 = _sc_scatter(tab_u1, tab_i1, s_ui, d_ui, s_iu, d_iu, zinit)
    x_item1 = _epi(acc_i1[:N], cnt_i[:N], r_item1, b1_ui, relu=True)
    x_user1 = _epi(acc_u1[:N], cnt_u[:N], r_user1, b1_iu, relu=True)

    tab_u2, r_user2 = _mm(x_user1, jnp.concatenate([W2_l_ui, W2_r_iu], axis=1))
    tab_i2, r_item2 = _mm(x_item1, jnp.concatenate([W2_l_iu, W2_r_ui], axis=1))
    acc_i2, acc_u2 = _sc_scatter(tab_u2, tab_i2, s_ui, d_ui, s_iu, d_iu, zinit)
    x_item2 = _epi(acc_i2[:N], cnt_i[:N], r_item2, b2_ui, relu=False)
    x_user2 = _epi(acc_u2[:N], cnt_u[:N], r_user2, b2_iu, relu=False)
    return (x_user2, x_item2)


# speedup vs baseline: 3.4373x; 1.1202x over previous
import jax
import jax.numpy as jnp
from jax import lax
from jax.experimental import pallas as pl
from jax.experimental.pallas import tpu as pltpu
from jax.experimental.pallas import tpu_sc as plsc

N = 10000
F = 128
N_PAD = 10112
NS = 16
NC = 2
CHUNK = 128
CHUNKS_PER_TILE = 80
E_PAD = NS * CHUNKS_PER_TILE * CHUNK
ROWS_PER_TILE = N_PAD // NS
BM = 400


def _mm_body(x_ref, w_ref, tab_ref, r_ref):
    z = jnp.dot(x_ref[...], w_ref[...], preferred_element_type=jnp.float32)
    tab_ref[...] = z[:, :F]
    r_ref[...] = z[:, F:]


def _mm(x, w_aug):
    k = x.shape[1]
    return pl.pallas_call(
        _mm_body,
        grid=(N // BM,),
        in_specs=[
            pl.BlockSpec((BM, k), lambda i: (i, 0)),
            pl.BlockSpec((k, 2 * F), lambda i: (0, 0)),
        ],
        out_specs=[
            pl.BlockSpec((BM, F), lambda i: (i, 0)),
            pl.BlockSpec((BM, F), lambda i: (i, 0)),
        ],
        out_shape=[
            jax.ShapeDtypeStruct((N, F), jnp.float32),
            jax.ShapeDtypeStruct((N, F), jnp.float32),
        ],
    )(x, w_aug)


def _epi_body_relu(acc_ref, cnt_ref, r_ref, b_ref, o_ref):
    _epi_common(acc_ref, cnt_ref, r_ref, b_ref, o_ref, True)


def _epi_body_lin(acc_ref, cnt_ref, r_ref, b_ref, o_ref):
    _epi_common(acc_ref, cnt_ref, r_ref, b_ref, o_ref, False)


def _epi_common(acc_ref, cnt_ref, r_ref, b_ref, o_ref, relu):
    cnt = jnp.maximum(cnt_ref[...], 1.0)
    out = acc_ref[...] / cnt + b_ref[...] + r_ref[...]
    if relu:
        out = jnp.maximum(out, 0.0)
    o_ref[...] = out


def _epi(acc, cnt, r, b, relu):
    body = _epi_body_relu if relu else _epi_body_lin
    return pl.pallas_call(
        body,
        grid=(N // BM,),
        in_specs=[
            pl.BlockSpec((BM, F), lambda i: (i, 0)),
            pl.BlockSpec((BM, F), lambda i: (i, 0)),
            pl.BlockSpec((BM, F), lambda i: (i, 0)),
            pl.BlockSpec((1, F), lambda i: (0, 0)),
        ],
        out_specs=pl.BlockSpec((BM, F), lambda i: (i, 0)),
        out_shape=jax.ShapeDtypeStruct((N, F), jnp.float32),
    )(acc, cnt, r, b.reshape(1, F))


NBUF = 2
KCH = 10
OUTER = CHUNKS_PER_TILE // KCH
NBUF_C = 10
OUTER_C = CHUNKS_PER_TILE // NBUF_C


def _sc_scatter_body(tab_u, tab_i, s_ui, d_ui, s_iu, d_iu, zinit,
                     out_i, out_u, src_b, dst_b, rows_b, acc_sh,
                     sem_i, sem_g, sem_s):
    cid = lax.axis_index("c")
    sid = lax.axis_index("s")
    row0 = sid * ROWS_PER_TILE
    pltpu.sync_copy(zinit, acc_sh.at[pl.ds(row0, ROWS_PER_TILE)])
    plsc.subcore_barrier()

    def run(tab, s_hbm, d_hbm):
        def outer(g0, carry):
            hs = []
            for j in range(KCH):
                base = sid * (CHUNKS_PER_TILE * CHUNK) \
                    + (g0 * KCH + j) * CHUNK
                h1 = pltpu.async_copy(s_hbm.at[pl.ds(base, CHUNK)],
                                      src_b[j], sem_i[j])
                h2 = pltpu.async_copy(d_hbm.at[pl.ds(base, CHUNK)],
                                      dst_b[j], sem_i[j])
                hs.append((h1, h2))
            gs = [None] * KCH
            ss = [None] * KCH
            for j in range(KCH):
                slot = j % NBUF
                if j >= NBUF:
                    ss[j - NBUF].wait()
                hs[j][0].wait()
                gs[j] = pltpu.async_copy(tab.at[src_b[j]],
                                         rows_b[slot], sem_g[slot])
                if j >= 1:
                    gs[j - 1].wait()
                    hs[j - 1][1].wait()
                    ss[j - 1] = pltpu.async_copy(
                        rows_b[(j - 1) % NBUF],
                        acc_sh.at[dst_b[j - 1]], sem_s[(j - 1) % NBUF],
                        add=True)
            gs[KCH - 1].wait()
            hs[KCH - 1][1].wait()
            ss[KCH - 1] = pltpu.async_copy(
                rows_b[(KCH - 1) % NBUF],
                acc_sh.at[dst_b[KCH - 1]], sem_s[(KCH - 1) % NBUF],
                add=True)
            ss[KCH - 2].wait()
            ss[KCH - 1].wait()
            return carry
        lax.fori_loop(0, OUTER, outer, 0)

    @pl.when(cid == 0)
    def _():
        run(tab_u, s_ui, d_ui)

    @pl.when(cid == 1)
    def _():
        run(tab_i, s_iu, d_iu)

    plsc.subcore_barrier()

    @pl.when(cid == 0)
    def _():
        pltpu.sync_copy(acc_sh.at[pl.ds(row0, ROWS_PER_TILE)],
                        out_i.at[pl.ds(row0, ROWS_PER_TILE)])

    @pl.when(cid == 1)
    def _():
        pltpu.sync_copy(acc_sh.at[pl.ds(row0, ROWS_PER_TILE)],
                        out_u.at[pl.ds(row0, ROWS_PER_TILE)])


def _sc_scatter(tab_u, tab_i, s_ui, d_ui, s_iu, d_iu, zinit):
    mesh = plsc.VectorSubcoreMesh(core_axis_name="c", subcore_axis_name="s",
                                  num_cores=NC, num_subcores=NS)
    f = pl.kernel(
        _sc_scatter_body,
        out_type=(jax.ShapeDtypeStruct((N_PAD, F), jnp.float32),
                  jax.ShapeDtypeStruct((N_PAD, F), jnp.float32)),
        mesh=mesh,
        scratch_types=[
            [pltpu.VMEM((CHUNK,), jnp.int32) for _ in range(KCH)],
            [pltpu.VMEM((CHUNK,), jnp.int32) for _ in range(KCH)],
            [pltpu.VMEM((CHUNK, F), jnp.float32) for _ in range(NBUF)],
            pltpu.VMEM_SHARED((N_PAD, F), jnp.float32),
            [pltpu.SemaphoreType.DMA for _ in range(KCH)],
            [pltpu.SemaphoreType.DMA for _ in range(NBUF)],
            [pltpu.SemaphoreType.DMA for _ in range(NBUF)],
        ],
    )
    return f(tab_u, tab_i, s_ui, d_ui, s_iu, d_iu, zinit)


def _sc_count_body(d_ui, d_iu, ones_rows, zinit, out_i, out_u,
                   dst_b, rows_v, acc_sh, sem_i, sem_s):
    cid = lax.axis_index("c")
    sid = lax.axis_index("s")
    row0 = sid * ROWS_PER_TILE
    pltpu.sync_copy(zinit, acc_sh.at[pl.ds(row0, ROWS_PER_TILE)])
    pltpu.sync_copy(ones_rows, rows_v)
    plsc.subcore_barrier()

    def run(d_hbm):
        def outer(g0, carry):
            hs = []
            for b in range(NBUF_C):
                base = sid * (CHUNKS_PER_TILE * CHUNK) \
                    + (g0 * NBUF_C + b) * CHUNK
                hs.append(pltpu.async_copy(d_hbm.at[pl.ds(base, CHUNK)],
                                           dst_b[b], sem_i[b]))
            ss = []
            for b in range(NBUF_C):
                hs[b].wait()
                ss.append(pltpu.async_copy(rows_v, acc_sh.at[dst_b[b]],
                                           sem_s[b], add=True))
            for b in range(NBUF_C):
                ss[b].wait()
            return carry
        lax.fori_loop(0, OUTER_C, outer, 0)

    @pl.when(cid == 0)
    def _():
        run(d_ui)

    @pl.when(cid == 1)
    def _():
        run(d_iu)

    plsc.subcore_barrier()

    @pl.when(cid == 0)
    def _():
        pltpu.sync_copy(acc_sh.at[pl.ds(row0, ROWS_PER_TILE)],
                        out_i.at[pl.ds(row0, ROWS_PER_TILE)])

    @pl.when(cid == 1)
    def _():
        pltpu.sync_copy(acc_sh.at[pl.ds(row0, ROWS_PER_TILE)],
                        out_u.at[pl.ds(row0, ROWS_PER_TILE)])


def _sc_count(d_ui, d_iu, ones_rows, zinit):
    mesh = plsc.VectorSubcoreMesh(core_axis_name="c", subcore_axis_name="s",
                                  num_cores=NC, num_subcores=NS)
    f = pl.kernel(
        _sc_count_body,
        out_type=(jax.ShapeDtypeStruct((N_PAD, F), jnp.float32),
                  jax.ShapeDtypeStruct((N_PAD, F), jnp.float32)),
        mesh=mesh,
        scratch_types=[
            [pltpu.VMEM((CHUNK,), jnp.int32) for _ in range(NBUF_C)],
            pltpu.VMEM((CHUNK, F), jnp.float32),
            pltpu.VMEM_SHARED((N_PAD, F), jnp.float32),
            [pltpu.SemaphoreType.DMA for _ in range(NBUF_C)],
            [pltpu.SemaphoreType.DMA for _ in range(NBUF_C)],
        ],
    )
    return f(d_ui, d_iu, ones_rows, zinit)


def _pad_edges(ei):
    src = ei[0].astype(jnp.int32)
    dst = ei[1].astype(jnp.int32)
    pad = E_PAD - src.shape[0]
    src = jnp.concatenate([src, jnp.zeros((pad,), jnp.int32)])
    dst = jnp.concatenate([dst, jnp.full((pad,), N, jnp.int32)])
    return src, dst


def kernel(x_user, x_item, edge_index_u2i, edge_index_i2u,
           W1_l_ui, b1_ui, W1_r_ui, W1_l_iu, b1_iu, W1_r_iu,
           W2_l_ui, b2_ui, W2_r_ui, W2_l_iu, b2_iu, W2_r_iu):
    s_ui, d_ui = _pad_edges(edge_index_u2i)
    s_iu, d_iu = _pad_edges(edge_index_i2u)
    zinit = jnp.zeros((ROWS_PER_TILE, F), jnp.float32)
    ones_rows = jnp.ones((CHUNK, F), jnp.float32)

    cnt_i, cnt_u = _sc_count(d_ui, d_iu, ones_rows, zinit)

    tab_u1, r_user1 = _mm(x_user, jnp.concatenate([W1_l_ui, W1_r_iu], axis=1))
    tab_i1, r_item1 = _mm(x_item, jnp.concatenate([W1_l_iu, W1_r_ui], axis=1))
    acc_i1, acc_u1 = _sc_scatter(tab_u1, tab_i1, s_ui, d_ui, s_iu, d_iu, zinit)
    x_item1 = _epi(acc_i1[:N], cnt_i[:N], r_item1, b1_ui, relu=True)
    x_user1 = _epi(acc_u1[:N], cnt_u[:N], r_user1, b1_iu, relu=True)

    tab_u2, r_user2 = _mm(x_user1, jnp.concatenate([W2_l_ui, W2_r_iu], axis=1))
    tab_i2, r_item2 = _mm(x_item1, jnp.concatenate([W2_l_iu, W2_r_ui], axis=1))
    acc_i2, acc_u2 = _sc_scatter(tab_u2, tab_i2, s_ui, d_ui, s_iu, d_iu, zinit)
    x_item2 = _epi(acc_i2[:N], cnt_i[:N], r_item2, b2_ui, relu=False)
    x_user2 = _epi(acc_u2[:N], cnt_u[:N], r_user2, b2_iu, relu=False)
    return (x_user2, x_item2)
